# baseline (speedup 1.0000x reference)
import numpy as np
import jax
import jax.numpy as jnp
from jax import lax
from jax.experimental import pallas as pl
from jax.experimental.pallas import tpu as pltpu
from jax.experimental.pallas import tpu_sc as plsc

_B = 4096
_H = 50
_D = 128
_NCLS = 1000
_NCLS_PAD = 1024

_NC, _NS = 2, 16
_NW = _NC * _NS
_EPT = _B // _NW
_RPT = _EPT * _H
_ECH = 2
_RCH = _ECH * _H
_NSTEP = _EPT // _ECH
_CG = _D // 16


_NBUF = 8
_UNROLL = 5


def _pool_body(idx_hbm, table_hbm, out_hbm, idx_v, *rest):
    bufs = rest[:_NBUF]
    out_v = rest[_NBUF]
    sems = rest[_NBUF + 1:]
    c = lax.axis_index("c")
    s = lax.axis_index("s")
    pltpu.sync_copy(idx_hbm.at[c, s], idx_v)

    def fire(t, k):
        pltpu.async_copy(table_hbm.at[idx_v.at[t]], bufs[k], sems[k])

    def drain(t, k):
        pltpu.make_async_copy(table_hbm.at[idx_v.at[t]], bufs[k],
                              sems[k]).wait()

    def reduce_chunk(t, buf):
        for e in range(_ECH):
            def rowadd(j, acc):
                base = e * _H + j * _UNROLL
                for u in range(_UNROLL):
                    acc = tuple(
                        acc[cc] + buf[base + u, cc * 16:(cc + 1) * 16]
                        for cc in range(_CG))
                return acc
            acc = lax.fori_loop(
                0, _H // _UNROLL, rowadd,
                tuple(jnp.zeros((16,), jnp.float32) for _ in range(_CG)))
            row = t * _ECH + e
            for cc in range(_CG):
                out_v[row, cc * 16:(cc + 1) * 16] = acc[cc]

    for k in range(_NBUF - 1):
        fire(k, k)

    def step4(i, _):
        t0 = _NBUF * i
        for k in range(_NBUF):
            t = t0 + k
            drain(t, k)

            @pl.when(t + _NBUF - 1 < _NSTEP)
            def _():
                fire(t + _NBUF - 1, (k + _NBUF - 1) % _NBUF)
            reduce_chunk(t, bufs[k])
        return _
    lax.fori_loop(0, _NSTEP // _NBUF, step4, None)

    pltpu.sync_copy(out_v, out_hbm.at[pl.ds((c * _NS + s) * _EPT, _EPT)])


def _pooled_sum(idx, table):
    mesh = plsc.VectorSubcoreMesh(core_axis_name="c", subcore_axis_name="s")
    kfn = pl.kernel(
        _pool_body,
        out_type=jax.ShapeDtypeStruct((_B, _D), jnp.float32),
        mesh=mesh,
        scratch_types=(
            [pltpu.VMEM((_NSTEP, _RCH), jnp.int32)]
            + [pltpu.VMEM((_RCH, _D), jnp.float32)
               for _ in range(_NBUF)]
            + [pltpu.VMEM((_EPT, _D), jnp.float32)]
            + [pltpu.SemaphoreType.DMA for _ in range(_NBUF)]
        ),
    )
    return kfn(idx, table)


def _head_body(x_ref, w_ref, b_ref, o_ref):
    x = x_ref[...] * (1.0 / _H)
    o_ref[...] = lax.dot_general(
        w_ref[...], x, (((1,), (1,)), ((), ())),
        preferred_element_type=jnp.float32) + b_ref[...]


def _head(pooled_sum, w, b2):
    m_blk = 1024
    return pl.pallas_call(
        _head_body,
        grid=(_B // m_blk,),
        in_specs=[
            pl.BlockSpec((m_blk, _D), lambda i: (i, 0)),
            pl.BlockSpec((_NCLS, _D), lambda i: (0, 0)),
            pl.BlockSpec((_NCLS, 1), lambda i: (0, 0)),
        ],
        out_specs=pl.BlockSpec((_NCLS, m_blk), lambda i: (0, i)),
        out_shape=jax.ShapeDtypeStruct((_NCLS, _B), jnp.float32),
    )(pooled_sum, w, b2)


def kernel(inputs, table, W, b):
    idx = inputs.astype(jnp.int32).reshape(_NC, _NS, _NSTEP, _RCH)
    pooled = _pooled_sum(idx, table)
    return _head(pooled, W, b.reshape(_NCLS, 1)).T


# speedup vs baseline: 1.0064x; 1.0064x over previous
import numpy as np
import jax
import jax.numpy as jnp
from jax import lax
from jax.experimental import pallas as pl
from jax.experimental.pallas import tpu as pltpu
from jax.experimental.pallas import tpu_sc as plsc

_B = 4096
_H = 50
_D = 128
_NCLS = 1000
_NCLS_PAD = 1024

_NC, _NS = 2, 16
_NW = _NC * _NS
_EPT = _B // _NW
_RPT = _EPT * _H
_ECH = 2
_RCH = _ECH * _H
_NSTEP = _EPT // _ECH
_CG = _D // 16


_NBUF = 4
_UNROLL = 5


def _pool_body(idx_hbm, table_hbm, out_hbm, idx_v, *rest):
    bufs = rest[:_NBUF]
    out_v = rest[_NBUF]
    sems = rest[_NBUF + 1:]
    c = lax.axis_index("c")
    s = lax.axis_index("s")
    pltpu.sync_copy(idx_hbm.at[c, s], idx_v)

    def fire(t, k):
        pltpu.async_copy(table_hbm.at[idx_v.at[t]], bufs[k], sems[k])

    def drain(t, k):
        pltpu.make_async_copy(table_hbm.at[idx_v.at[t]], bufs[k],
                              sems[k]).wait()

    def reduce_chunk(t, buf):
        for e in range(_ECH):
            def rowadd(j, acc):
                base = e * _H + j * _UNROLL
                for u in range(_UNROLL):
                    acc = tuple(
                        acc[cc] + buf[base + u, cc * 16:(cc + 1) * 16]
                        for cc in range(_CG))
                return acc
            acc = lax.fori_loop(
                0, _H // _UNROLL, rowadd,
                tuple(jnp.zeros((16,), jnp.float32) for _ in range(_CG)))
            row = t * _ECH + e
            for cc in range(_CG):
                out_v[row, cc * 16:(cc + 1) * 16] = acc[cc]

    for k in range(_NBUF - 1):
        fire(k, k)

    def step4(i, _):
        t0 = _NBUF * i
        for k in range(_NBUF):
            t = t0 + k
            drain(t, k)

            @pl.when(t + _NBUF - 1 < _NSTEP)
            def _():
                fire(t + _NBUF - 1, (k + _NBUF - 1) % _NBUF)
            reduce_chunk(t, bufs[k])
        return _
    lax.fori_loop(0, _NSTEP // _NBUF, step4, None)

    pltpu.sync_copy(out_v, out_hbm.at[pl.ds((c * _NS + s) * _EPT, _EPT)])


def _pooled_sum(idx, table):
    mesh = plsc.VectorSubcoreMesh(core_axis_name="c", subcore_axis_name="s")
    kfn = pl.kernel(
        _pool_body,
        out_type=jax.ShapeDtypeStruct((_B, _D), jnp.float32),
        mesh=mesh,
        scratch_types=(
            [pltpu.VMEM((_NSTEP, _RCH), jnp.int32)]
            + [pltpu.VMEM((_RCH, _D), jnp.float32)
               for _ in range(_NBUF)]
            + [pltpu.VMEM((_EPT, _D), jnp.float32)]
            + [pltpu.SemaphoreType.DMA for _ in range(_NBUF)]
        ),
    )
    return kfn(idx, table)


def _head_body(x_ref, w_ref, b_ref, o_ref):
    x = x_ref[...] * (1.0 / _H)
    o_ref[...] = lax.dot_general(
        w_ref[...], x, (((1,), (1,)), ((), ())),
        preferred_element_type=jnp.float32) + b_ref[...]


def _head(pooled_sum, w, b2):
    m_blk = 512
    return pl.pallas_call(
        _head_body,
        grid=(_B // m_blk,),
        in_specs=[
            pl.BlockSpec((m_blk, _D), lambda i: (i, 0)),
            pl.BlockSpec((_NCLS, _D), lambda i: (0, 0)),
            pl.BlockSpec((_NCLS, 1), lambda i: (0, 0)),
        ],
        out_specs=pl.BlockSpec((_NCLS, m_blk), lambda i: (0, i)),
        out_shape=jax.ShapeDtypeStruct((_NCLS, _B), jnp.float32),
    )(pooled_sum, w, b2)


def kernel(inputs, table, W, b):
    idx = inputs.astype(jnp.int32).reshape(_NC, _NS, _NSTEP, _RCH)
    pooled = _pooled_sum(idx, table)
    return _head(pooled, W, b.reshape(_NCLS, 1)).T


# speedup vs baseline: 1.0280x; 1.0215x over previous
import numpy as np
import jax
import jax.numpy as jnp
from jax import lax
from jax.experimental import pallas as pl
from jax.experimental.pallas import tpu as pltpu
from jax.experimental.pallas import tpu_sc as plsc

_B = 4096
_H = 50
_D = 128
_NCLS = 1000
_NCLS_PAD = 1024

_NC, _NS = 2, 16
_NW = _NC * _NS
_EPT = _B // _NW
_RPT = _EPT * _H
_ECH = 2
_RCH = _ECH * _H
_NSTEP = _EPT // _ECH
_CG = _D // 16


_NBUF = 4
_UNROLL = 5


def _pool_body(idx_hbm, table_hbm, out_hbm, idx_v, *rest):
    bufs = rest[:_NBUF]
    out_v = rest[_NBUF]
    sems = rest[_NBUF + 1:]
    c = lax.axis_index("c")
    s = lax.axis_index("s")
    pltpu.sync_copy(idx_hbm.at[c, s], idx_v)

    def fire(t, k):
        pltpu.async_copy(table_hbm.at[idx_v.at[t]], bufs[k], sems[k])

    def drain(t, k):
        pltpu.make_async_copy(table_hbm.at[idx_v.at[t]], bufs[k],
                              sems[k]).wait()

    def reduce_chunk(t, buf):
        for e in range(_ECH):
            def rowadd(j, acc):
                base = e * _H + j * _UNROLL
                for u in range(_UNROLL):
                    acc = tuple(
                        acc[cc] + buf[base + u, cc * 16:(cc + 1) * 16]
                        for cc in range(_CG))
                return acc
            acc = lax.fori_loop(
                0, _H // _UNROLL, rowadd,
                tuple(jnp.zeros((16,), jnp.float32) for _ in range(_CG)))
            row = t * _ECH + e
            for cc in range(_CG):
                out_v[row, cc * 16:(cc + 1) * 16] = acc[cc]

    for k in range(_NBUF - 1):
        fire(k, k)

    def step4(i, _):
        t0 = _NBUF * i
        for k in range(_NBUF):
            t = t0 + k
            drain(t, k)

            @pl.when(t + _NBUF - 1 < _NSTEP)
            def _():
                fire(t + _NBUF - 1, (k + _NBUF - 1) % _NBUF)
            reduce_chunk(t, bufs[k])
        return _
    lax.fori_loop(0, _NSTEP // _NBUF, step4, None)

    pltpu.sync_copy(out_v, out_hbm.at[pl.ds((c * _NS + s) * _EPT, _EPT)])


def _pooled_sum(idx, table):
    mesh = plsc.VectorSubcoreMesh(core_axis_name="c", subcore_axis_name="s")
    kfn = pl.kernel(
        _pool_body,
        out_type=jax.ShapeDtypeStruct((_B, _D), jnp.float32),
        mesh=mesh,
        scratch_types=(
            [pltpu.VMEM((_NSTEP, _RCH), jnp.int32)]
            + [pltpu.VMEM((_RCH, _D), jnp.float32)
               for _ in range(_NBUF)]
            + [pltpu.VMEM((_EPT, _D), jnp.float32)]
            + [pltpu.SemaphoreType.DMA for _ in range(_NBUF)]
        ),
    )
    return kfn(idx, table)


def _head_body(x_ref, w_ref, b_ref, o_ref):
    x = x_ref[...] * (1.0 / _H)
    o_ref[...] = lax.dot_general(
        w_ref[...], x, (((1,), (1,)), ((), ())),
        preferred_element_type=jnp.float32) + b_ref[...]


def _head(pooled_sum, w, b2):
    m_blk = 2048
    return pl.pallas_call(
        _head_body,
        grid=(_B // m_blk,),
        in_specs=[
            pl.BlockSpec((m_blk, _D), lambda i: (i, 0)),
            pl.BlockSpec((_NCLS, _D), lambda i: (0, 0)),
            pl.BlockSpec((_NCLS, 1), lambda i: (0, 0)),
        ],
        out_specs=pl.BlockSpec((_NCLS, m_blk), lambda i: (0, i)),
        out_shape=jax.ShapeDtypeStruct((_NCLS, _B), jnp.float32),
    )(pooled_sum, w, b2)


def kernel(inputs, table, W, b):
    idx = inputs.astype(jnp.int32).reshape(_NC, _NS, _NSTEP, _RCH)
    pooled = _pooled_sum(idx, table)
    return _head(pooled, W, b.reshape(_NCLS, 1)).T
